# baseline (speedup 1.0000x reference)
import functools

import jax
import jax.numpy as jnp
from jax import lax
from jax.experimental import pallas as pl
from jax.experimental.pallas import tpu as pltpu
from jax.experimental.pallas import tpu_sc as plsc

VOCAB = 1000000
CHANNELS = 32
B = 16384
L = 50
NUM_CORES = 2
NUM_SUBCORES = 16
NUM_WORKERS = NUM_CORES * NUM_SUBCORES
TPW = 4
TOK = TPW * 128

_mesh = plsc.VectorSubcoreMesh(core_axis_name="c", subcore_axis_name="s")


@functools.partial(
    pl.kernel,
    mesh=_mesh,
    compiler_params=pltpu.CompilerParams(use_tc_tiling_on_sc=False,
                                         needs_layout_passes=False),
    out_type=jax.ShapeDtypeStruct((L, CHANNELS // 8, B // 128, 8, 128),
                                  jnp.float32),
    scratch_types=[
        pltpu.VMEM((L, TOK), jnp.int32),
        pltpu.VMEM((TOK, CHANNELS), jnp.float32),
        pltpu.VMEM((TOK, CHANNELS), jnp.float32),
        pltpu.VMEM((CHANNELS // 8, TPW, 8, 128), jnp.float32),
        pltpu.VMEM((CHANNELS // 8, TPW, 8, 128), jnp.float32),
        pltpu.SemaphoreType.DMA,
        pltpu.SemaphoreType.DMA,
        pltpu.SemaphoreType.DMA,
        pltpu.SemaphoreType.DMA,
    ],
)
def _embed(xt_hbm, table_hbm, out_hbm, idxb, rows0, rows1, tiles0, tiles1,
           gsem0, gsem1, osem0, osem1):
    w = lax.axis_index("s") * NUM_CORES + lax.axis_index("c")
    iota = lax.iota(jnp.int32, 16)

    pltpu.sync_copy(xt_hbm.at[:, pl.ds(TOK * w, TOK)], idxb)

    def gather_start(l, rows, gsem):
        return pltpu.async_copy(table_hbm.at[idxb.at[l]], rows, gsem)

    def gather_wait(l, rows, gsem):
        pltpu.make_async_copy(table_hbm.at[idxb.at[l]], rows, gsem).wait()

    ridx_tab = [[iota + (tp * 128 + 16 * k) for k in range(8)]
                for tp in range(TPW)]

    def transpose(rows, tiles):
        @plsc.parallel_loop(0, CHANNELS, unroll=8)
        def _(c):
            tr = c // 8
            s = c - tr * 8
            cvec = jnp.broadcast_to(c, (16,)).astype(jnp.int32)
            for tp in range(TPW):
                for k in range(8):
                    v = plsc.load_gather(rows, [ridx_tab[tp][k], cvec])
                    tiles[tr, tp, s, pl.ds(16 * k, 16)] = v

    def out_start(l, tiles, osem):
        return pltpu.async_copy(
            tiles, out_hbm.at[l, :, pl.ds(TPW * w, TPW)], osem)

    def out_wait(l, tiles, osem):
        pltpu.make_async_copy(
            tiles, out_hbm.at[l, :, pl.ds(TPW * w, TPW)], osem).wait()

    gather_start(0, rows0, gsem0)

    def pair(i, carry):
        l0 = 2 * i
        l1 = l0 + 1
        gather_wait(l0, rows0, gsem0)
        gather_start(l1, rows1, gsem1)

        @pl.when(i > 0)
        def _():
            out_wait(l0, tiles0, osem0)

        transpose(rows0, tiles0)
        out_start(l0, tiles0, osem0)

        gather_wait(l1, rows1, gsem1)

        @pl.when(i < (L // 2) - 1)
        def _():
            gather_start(l1 + 1, rows0, gsem0)

        @pl.when(i > 0)
        def _():
            out_wait(l1, tiles1, osem1)

        transpose(rows1, tiles1)
        out_start(l1, tiles1, osem1)
        return carry

    lax.fori_loop(0, L // 2, pair, 0)

    out_wait(L - 2, tiles0, osem0)
    out_wait(L - 1, tiles1, osem1)


def kernel(x, table):
    xt = x.T.astype(jnp.int32)
    out5 = _embed(xt, table)
    return jnp.transpose(out5, (2, 4, 0, 1, 3)).reshape(B, L, CHANNELS)


# speedup vs baseline: 1.0212x; 1.0212x over previous
import functools

import jax
import jax.numpy as jnp
from jax import lax
from jax.experimental import pallas as pl
from jax.experimental.pallas import tpu as pltpu
from jax.experimental.pallas import tpu_sc as plsc

VOCAB = 1000000
CHANNELS = 32
B = 16384
L = 50
NUM_CORES = 2
NUM_SUBCORES = 16
NUM_WORKERS = NUM_CORES * NUM_SUBCORES
TPW = 4
TOK = TPW * 128

_mesh = plsc.VectorSubcoreMesh(core_axis_name="c", subcore_axis_name="s")


@functools.partial(
    pl.kernel,
    mesh=_mesh,
    compiler_params=pltpu.CompilerParams(use_tc_tiling_on_sc=False,
                                         needs_layout_passes=False),
    out_type=jax.ShapeDtypeStruct((L, CHANNELS // 8, B // 128, 8, 128),
                                  jnp.float32),
    scratch_types=[
        pltpu.VMEM((L, TOK), jnp.int32),
        pltpu.VMEM((TOK, CHANNELS), jnp.float32),
        pltpu.VMEM((TOK, CHANNELS), jnp.float32),
        pltpu.VMEM((CHANNELS // 8, TPW, 8, 128), jnp.float32),
        pltpu.VMEM((CHANNELS // 8, TPW, 8, 128), jnp.float32),
        pltpu.SemaphoreType.DMA,
        pltpu.SemaphoreType.DMA,
        pltpu.SemaphoreType.DMA,
        pltpu.SemaphoreType.DMA,
    ],
)
def _embed(xt_hbm, table_hbm, out_hbm, idxb, rows0, rows1, tiles0, tiles1,
           gsem0, gsem1, osem0, osem1):
    w = lax.axis_index("s") * NUM_CORES + lax.axis_index("c")
    iota = lax.iota(jnp.int32, 16)

    pltpu.sync_copy(xt_hbm.at[:, pl.ds(TOK * w, TOK)], idxb)

    def gather_start(l, rows, gsem):
        return pltpu.async_copy(table_hbm.at[idxb.at[l]], rows, gsem)

    def gather_wait(l, rows, gsem):
        pltpu.make_async_copy(table_hbm.at[idxb.at[l]], rows, gsem).wait()

    ridx_tab = [[iota + (tp * 128 + 16 * k) for k in range(8)]
                for tp in range(TPW)]

    def transpose(rows, tiles):
        @plsc.parallel_loop(0, CHANNELS * TPW, unroll=8)
        def _(i):
            c = i // TPW
            tp = i - c * TPW
            tr = c // 8
            s = c - tr * 8
            cvec = jnp.broadcast_to(c, (16,)).astype(jnp.int32)
            base = tp * 128
            for k in range(8):
                ridx = iota + (base + 16 * k)
                v = plsc.load_gather(rows, [ridx, cvec])
                tiles[tr, tp, s, pl.ds(16 * k, 16)] = v

    def out_start(l, tiles, osem):
        return pltpu.async_copy(
            tiles, out_hbm.at[l, :, pl.ds(TPW * w, TPW)], osem)

    def out_wait(l, tiles, osem):
        pltpu.make_async_copy(
            tiles, out_hbm.at[l, :, pl.ds(TPW * w, TPW)], osem).wait()

    gather_start(0, rows0, gsem0)

    def pair(i, carry):
        l0 = 2 * i
        l1 = l0 + 1
        gather_wait(l0, rows0, gsem0)
        gather_start(l1, rows1, gsem1)

        @pl.when(i > 0)
        def _():
            out_wait(l0, tiles0, osem0)

        transpose(rows0, tiles0)
        out_start(l0, tiles0, osem0)

        gather_wait(l1, rows1, gsem1)

        @pl.when(i < (L // 2) - 1)
        def _():
            gather_start(l1 + 1, rows0, gsem0)

        @pl.when(i > 0)
        def _():
            out_wait(l1, tiles1, osem1)

        transpose(rows1, tiles1)
        out_start(l1, tiles1, osem1)
        return carry

    lax.fori_loop(0, L // 2, pair, 0)

    out_wait(L - 2, tiles0, osem0)
    out_wait(L - 1, tiles1, osem1)


def kernel(x, table):
    xt = x.T.astype(jnp.int32)
    out5 = _embed(xt, table)
    return jnp.transpose(out5, (2, 4, 0, 1, 3)).reshape(B, L, CHANNELS)


# speedup vs baseline: 1.5252x; 1.4935x over previous
import functools

import jax
import jax.numpy as jnp
from jax import lax
from jax.experimental import pallas as pl
from jax.experimental.pallas import tpu as pltpu
from jax.experimental.pallas import tpu_sc as plsc

VOCAB = 1000000
CHANNELS = 32
B = 16384
L = 50
NUM_CORES = 2
NUM_SUBCORES = 16
NUM_WORKERS = NUM_CORES * NUM_SUBCORES
TPW = 4
TOK = TPW * 128

_mesh = plsc.VectorSubcoreMesh(core_axis_name="c", subcore_axis_name="s")


@functools.partial(
    pl.kernel,
    mesh=_mesh,
    compiler_params=pltpu.CompilerParams(use_tc_tiling_on_sc=False,
                                         needs_layout_passes=False),
    out_type=jax.ShapeDtypeStruct((L, CHANNELS // 8, B // 128, 8, 128),
                                  jnp.float32),
    scratch_types=[
        pltpu.VMEM((L, TOK), jnp.int32),
        pltpu.VMEM((TOK, CHANNELS), jnp.float32),
        pltpu.VMEM((TOK, CHANNELS), jnp.float32),
        pltpu.VMEM((CHANNELS // 8, TPW, 8, 136), jnp.float32),
        pltpu.VMEM((CHANNELS // 8, TPW, 8, 136), jnp.float32),
        pltpu.SemaphoreType.DMA,
        pltpu.SemaphoreType.DMA,
        pltpu.SemaphoreType.DMA,
        pltpu.SemaphoreType.DMA,
    ],
)
def _embed(xt_hbm, table_hbm, out_hbm, idxb, rows0, rows1, tiles0, tiles1,
           gsem0, gsem1, osem0, osem1):
    w = lax.axis_index("s") * NUM_CORES + lax.axis_index("c")
    iota = lax.iota(jnp.int32, 16)

    pltpu.sync_copy(xt_hbm.at[:, pl.ds(TOK * w, TOK)], idxb)

    def gather_start(l, rows, gsem):
        return pltpu.async_copy(table_hbm.at[idxb.at[l]], rows, gsem)

    def gather_wait(l, rows, gsem):
        pltpu.make_async_copy(table_hbm.at[idxb.at[l]], rows, gsem).wait()

    trv = [(iota + 16 * h) // 8 for h in range(2)]
    sv = [(iota + 16 * h) % 8 for h in range(2)]

    def transpose(rows, tiles):
        @plsc.parallel_loop(0, TOK, unroll=4)
        def _(r):
            tp = r // 128
            lane = r - tp * 128
            tpv = jnp.broadcast_to(tp, (16,)).astype(jnp.int32)
            lanev = jnp.broadcast_to(lane, (16,)).astype(jnp.int32)
            for h in range(2):
                v = rows[r, pl.ds(16 * h, 16)]
                plsc.store_scatter(tiles, [trv[h], tpv, sv[h], lanev], v)

    def out_start(l, tiles, osem):
        return pltpu.async_copy(
            tiles.at[:, :, :, pl.ds(0, 128)],
            out_hbm.at[l, :, pl.ds(TPW * w, TPW)], osem)

    def out_wait(l, tiles, osem):
        pltpu.make_async_copy(
            tiles.at[:, :, :, pl.ds(0, 128)],
            out_hbm.at[l, :, pl.ds(TPW * w, TPW)], osem).wait()

    gather_start(0, rows0, gsem0)

    def pair(i, carry):
        l0 = 2 * i
        l1 = l0 + 1
        gather_wait(l0, rows0, gsem0)
        gather_start(l1, rows1, gsem1)

        @pl.when(i > 0)
        def _():
            out_wait(l0, tiles0, osem0)

        transpose(rows0, tiles0)
        out_start(l0, tiles0, osem0)

        gather_wait(l1, rows1, gsem1)

        @pl.when(i < (L // 2) - 1)
        def _():
            gather_start(l1 + 1, rows0, gsem0)

        @pl.when(i > 0)
        def _():
            out_wait(l1, tiles1, osem1)

        transpose(rows1, tiles1)
        out_start(l1, tiles1, osem1)
        return carry

    lax.fori_loop(0, L // 2, pair, 0)

    out_wait(L - 2, tiles0, osem0)
    out_wait(L - 1, tiles1, osem1)


def kernel(x, table):
    xt = x.T.astype(jnp.int32)
    out5 = _embed(xt, table)
    return jnp.transpose(out5, (2, 4, 0, 1, 3)).reshape(B, L, CHANNELS)


# speedup vs baseline: 1.5256x; 1.0003x over previous
import functools

import jax
import jax.numpy as jnp
from jax import lax
from jax.experimental import pallas as pl
from jax.experimental.pallas import tpu as pltpu
from jax.experimental.pallas import tpu_sc as plsc

VOCAB = 1000000
CHANNELS = 32
B = 16384
L = 50
NUM_CORES = 2
NUM_SUBCORES = 16
NUM_WORKERS = NUM_CORES * NUM_SUBCORES
TPW = 4
TOK = TPW * 128

_mesh = plsc.VectorSubcoreMesh(core_axis_name="c", subcore_axis_name="s")


@functools.partial(
    pl.kernel,
    mesh=_mesh,
    compiler_params=pltpu.CompilerParams(use_tc_tiling_on_sc=False,
                                         needs_layout_passes=False,
                                         skip_device_barrier=True,
                                         disable_bounds_checks=True,
                                         disable_semaphore_checks=True),
    out_type=jax.ShapeDtypeStruct((L, CHANNELS // 8, B // 128, 8, 128),
                                  jnp.float32),
    scratch_types=[
        pltpu.VMEM((L, TOK), jnp.int32),
        pltpu.VMEM((TOK, CHANNELS), jnp.float32),
        pltpu.VMEM((TOK, CHANNELS), jnp.float32),
        pltpu.VMEM((CHANNELS // 8, TPW, 8, 136), jnp.float32),
        pltpu.VMEM((CHANNELS // 8, TPW, 8, 136), jnp.float32),
        pltpu.SemaphoreType.DMA,
        pltpu.SemaphoreType.DMA,
        pltpu.SemaphoreType.DMA,
        pltpu.SemaphoreType.DMA,
    ],
)
def _embed(xt_hbm, table_hbm, out_hbm, idxb, rows0, rows1, tiles0, tiles1,
           gsem0, gsem1, osem0, osem1):
    w = lax.axis_index("s") * NUM_CORES + lax.axis_index("c")
    iota = lax.iota(jnp.int32, 16)

    pltpu.sync_copy(xt_hbm.at[:, pl.ds(TOK * w, TOK)], idxb)

    def gather_start(l, rows, gsem):
        return pltpu.async_copy(table_hbm.at[idxb.at[l]], rows, gsem)

    def gather_wait(l, rows, gsem):
        pltpu.make_async_copy(table_hbm.at[idxb.at[l]], rows, gsem).wait()

    trv = [(iota + 16 * h) // 8 for h in range(2)]
    sv = [(iota + 16 * h) % 8 for h in range(2)]

    def transpose(rows, tiles):
        @plsc.parallel_loop(0, TOK, unroll=4)
        def _(r):
            tp = r // 128
            lane = r - tp * 128
            tpv = jnp.broadcast_to(tp, (16,)).astype(jnp.int32)
            lanev = jnp.broadcast_to(lane, (16,)).astype(jnp.int32)
            for h in range(2):
                v = rows[r, pl.ds(16 * h, 16)]
                plsc.store_scatter(tiles, [trv[h], tpv, sv[h], lanev], v)

    def out_start(l, tiles, osem):
        return pltpu.async_copy(
            tiles.at[:, :, :, pl.ds(0, 128)],
            out_hbm.at[l, :, pl.ds(TPW * w, TPW)], osem)

    def out_wait(l, tiles, osem):
        pltpu.make_async_copy(
            tiles.at[:, :, :, pl.ds(0, 128)],
            out_hbm.at[l, :, pl.ds(TPW * w, TPW)], osem).wait()

    gather_start(0, rows0, gsem0)

    def pair(i, carry):
        l0 = 2 * i
        l1 = l0 + 1
        gather_wait(l0, rows0, gsem0)
        gather_start(l1, rows1, gsem1)

        @pl.when(i > 0)
        def _():
            out_wait(l0, tiles0, osem0)

        transpose(rows0, tiles0)
        out_start(l0, tiles0, osem0)

        gather_wait(l1, rows1, gsem1)

        @pl.when(i < (L // 2) - 1)
        def _():
            gather_start(l1 + 1, rows0, gsem0)

        @pl.when(i > 0)
        def _():
            out_wait(l1, tiles1, osem1)

        transpose(rows1, tiles1)
        out_start(l1, tiles1, osem1)
        return carry

    lax.fori_loop(0, L // 2, pair, 0)

    out_wait(L - 2, tiles0, osem0)
    out_wait(L - 1, tiles1, osem1)


def kernel(x, table):
    xt = x.T.astype(jnp.int32)
    out5 = _embed(xt, table)
    return jnp.transpose(out5, (2, 4, 0, 1, 3)).reshape(B, L, CHANNELS)
